# Initial kernel scaffold; baseline (speedup 1.0000x reference)
import jax
import jax.numpy as jnp
from jax.experimental import pallas as pl


def kernel(x, edge_index, W1, b1, W2, b2, Wf1, bf1, Wf2, bf2):
    raise NotImplementedError("write your pallas kernel here")


# speedup vs baseline: 12.7419x; 12.7419x over previous
import functools

import jax
import jax.numpy as jnp
from jax import lax
from jax.experimental import pallas as pl
from jax.experimental.pallas import tpu as pltpu
from jax.experimental.pallas import tpu_sc as plsc

_N = 10000
_E = 320000
_D = 128

_NC = 2
_NS = 16
_CHUNK = 80
_EDGES_PER_TILE = _E // (_NC * _NS)
_STEPS = _EDGES_PER_TILE // _CHUNK
_N_PAD = 10240
_ROWS_PER_TILE = _N_PAD // _NS
_DEG_W = 8


def _sc_degree_body(dst_hbm, ones_hbm, zeros_hbm, out_hbm, didx, ones_v, acc, sem):
    c = lax.axis_index("c")
    s = lax.axis_index("s")
    tile_base = (c * _NS + s) * _EDGES_PER_TILE
    row_base = s * _ROWS_PER_TILE
    pltpu.sync_copy(zeros_hbm.at[pl.ds(row_base, _ROWS_PER_TILE)],
                    acc.at[pl.ds(row_base, _ROWS_PER_TILE)])
    pltpu.sync_copy(ones_hbm, ones_v)
    plsc.subcore_barrier()

    def step(i, carry):
        e0 = pl.multiple_of(tile_base + i * _CHUNK, _CHUNK)
        pltpu.sync_copy(dst_hbm.at[pl.ds(e0, _CHUNK)], didx)
        pltpu.sync_copy(ones_v, acc.at[didx], add=True)
        return carry

    lax.fori_loop(0, _STEPS, step, 0)
    plsc.subcore_barrier()
    pltpu.sync_copy(acc.at[pl.ds(row_base, _ROWS_PER_TILE)],
                    out_hbm.at[c, pl.ds(row_base, _ROWS_PER_TILE)])


@functools.lru_cache(maxsize=None)
def _get_sc_degree():
    mesh = plsc.VectorSubcoreMesh(core_axis_name="c", subcore_axis_name="s")
    return pl.kernel(
        _sc_degree_body,
        out_type=jax.ShapeDtypeStruct((_NC, _N_PAD, _DEG_W), jnp.float32),
        mesh=mesh,
        scratch_types=[
            pltpu.VMEM((_CHUNK,), jnp.int32),
            pltpu.VMEM((_CHUNK, _DEG_W), jnp.float32),
            pltpu.VMEM_SHARED((_N_PAD, _DEG_W), jnp.float32),
            pltpu.SemaphoreType.DMA,
        ],
    )


def _sc_scatter_body(hp_hbm, src_hbm, dst_hbm, zeros_hbm, out_hbm,
                     sidx, didx, rows, acc, sem):
    c = lax.axis_index("c")
    s = lax.axis_index("s")
    tile_base = (c * _NS + s) * _EDGES_PER_TILE
    row_base = s * _ROWS_PER_TILE
    pltpu.sync_copy(zeros_hbm.at[pl.ds(row_base, _ROWS_PER_TILE)],
                    acc.at[pl.ds(row_base, _ROWS_PER_TILE)])
    plsc.subcore_barrier()

    def step(i, carry):
        e0 = pl.multiple_of(tile_base + i * _CHUNK, _CHUNK)
        pltpu.sync_copy(src_hbm.at[pl.ds(e0, _CHUNK)], sidx)
        pltpu.sync_copy(dst_hbm.at[pl.ds(e0, _CHUNK)], didx)
        pltpu.async_copy(hp_hbm.at[sidx], rows, sem).wait()
        pltpu.sync_copy(rows, acc.at[didx], add=True)
        return carry

    lax.fori_loop(0, _STEPS, step, 0)
    plsc.subcore_barrier()
    pltpu.sync_copy(acc.at[pl.ds(row_base, _ROWS_PER_TILE)],
                    out_hbm.at[c, pl.ds(row_base, _ROWS_PER_TILE)])


@functools.lru_cache(maxsize=None)
def _get_sc_scatter():
    mesh = plsc.VectorSubcoreMesh(core_axis_name="c", subcore_axis_name="s")
    return pl.kernel(
        _sc_scatter_body,
        out_type=jax.ShapeDtypeStruct((_NC, _N_PAD, _D), jnp.float32),
        mesh=mesh,
        scratch_types=[
            pltpu.VMEM((_CHUNK,), jnp.int32),
            pltpu.VMEM((_CHUNK,), jnp.int32),
            pltpu.VMEM((_CHUNK, _D), jnp.float32),
            pltpu.VMEM_SHARED((_N_PAD, _D), jnp.float32),
            pltpu.SemaphoreType.DMA,
        ],
    )


_BLK = 400
_GRID = _N // _BLK


def _tc1_body(degp_ref, x_ref, w1_ref, hp_ref, dinv_ref):
    deg = degp_ref[0, :, 0:1] + degp_ref[1, :, 0:1] + 1.0
    dinv = lax.rsqrt(deg)
    h = jnp.dot(x_ref[...], w1_ref[...], preferred_element_type=jnp.float32)
    hp_ref[...] = dinv * h
    dinv_ref[...] = dinv


def _tc2_body(parts_ref, hp_ref, dinv_ref, b1_ref, w2_ref, hp2_ref):
    dinv = dinv_ref[...]
    agg = parts_ref[0] + parts_ref[1] + hp_ref[...]
    o1 = jnp.maximum(dinv * agg + b1_ref[...], 0.0)
    hp2_ref[...] = dinv * jnp.dot(o1, w2_ref[...], preferred_element_type=jnp.float32)


def _tc3_body(parts_ref, hp_ref, dinv_ref, b2_ref, wf1_ref, bf1_ref,
              wf2_ref, bf2_ref, y_ref):
    dinv = dinv_ref[...]
    agg = parts_ref[0] + parts_ref[1] + hp_ref[...]
    o2 = jnp.maximum(dinv * agg + b2_ref[...], 0.0)
    h3 = jnp.maximum(
        jnp.dot(o2, wf1_ref[...], preferred_element_type=jnp.float32) + bf1_ref[...],
        0.0)
    y_ref[...] = jnp.dot(h3, wf2_ref[...], preferred_element_type=jnp.float32) + bf2_ref[...]


def _row_blk(*trail):
    return pl.BlockSpec((_BLK,) + trail, lambda i: (i,) + (0,) * len(trail))


def _parts_blk(width):
    return pl.BlockSpec((_NC, _BLK, width), lambda i: (0, i, 0))


def _full(shape):
    return pl.BlockSpec(shape, lambda i: (0,) * len(shape))


_tc1 = pl.pallas_call(
    _tc1_body,
    grid=(_GRID,),
    in_specs=[_parts_blk(_DEG_W), _row_blk(_D), _full((_D, _D))],
    out_specs=[_row_blk(_D), _row_blk(1)],
    out_shape=[jax.ShapeDtypeStruct((_N, _D), jnp.float32),
               jax.ShapeDtypeStruct((_N, 1), jnp.float32)],
)

_tc2 = pl.pallas_call(
    _tc2_body,
    grid=(_GRID,),
    in_specs=[_parts_blk(_D), _row_blk(_D), _row_blk(1), _full((1, _D)),
              _full((_D, _D))],
    out_specs=_row_blk(_D),
    out_shape=jax.ShapeDtypeStruct((_N, _D), jnp.float32),
)

_tc3 = pl.pallas_call(
    _tc3_body,
    grid=(_GRID,),
    in_specs=[_parts_blk(_D), _row_blk(_D), _row_blk(1), _full((1, _D)),
              _full((_D, 64)), _full((1, 64)), _full((64, 1)), _full((1, 1))],
    out_specs=_row_blk(1),
    out_shape=jax.ShapeDtypeStruct((_N, 1), jnp.float32),
)


def kernel(x, edge_index, W1, b1, W2, b2, Wf1, bf1, Wf2, bf2):
    src = edge_index[0]
    dst = edge_index[1]
    zeros_deg = jnp.zeros((_N_PAD, _DEG_W), jnp.float32)
    zeros_big = jnp.zeros((_N_PAD, _D), jnp.float32)
    ones_rows = jnp.ones((_CHUNK, _DEG_W), jnp.float32)

    deg_parts = _get_sc_degree()(dst, ones_rows, zeros_deg)
    hp1, dinv = _tc1(deg_parts, x, W1)
    a1 = _get_sc_scatter()(hp1, src, dst, zeros_big)
    hp2 = _tc2(a1, hp1, dinv, b1.reshape(1, _D), W2)
    a2 = _get_sc_scatter()(hp2, src, dst, zeros_big)
    y = _tc3(a2, hp2, dinv, b2.reshape(1, _D), Wf1, bf1.reshape(1, 64),
             Wf2, bf2.reshape(1, 1))
    return y


# speedup vs baseline: 26.7464x; 2.0991x over previous
import functools

import jax
import jax.numpy as jnp
from jax import lax
from jax.experimental import pallas as pl
from jax.experimental.pallas import tpu as pltpu
from jax.experimental.pallas import tpu_sc as plsc

_N = 10000
_E = 320000
_D = 128

_NC = 2
_NS = 16
_CHUNK = 80
_EDGES_PER_TILE = _E // (_NC * _NS)
_STEPS = _EDGES_PER_TILE // _CHUNK
_N_PAD = 10240
_ROWS_PER_TILE = _N_PAD // _NS
_DEG_W = 8
_BLK_CH = 40
_NFULL = _STEPS // _BLK_CH
_TAIL = _STEPS - _NFULL * _BLK_CH


def _sc_degree_body(dst_hbm, ones_hbm, zeros_hbm, out_hbm, didx, ones_v, acc, sem):
    c = lax.axis_index("c")
    s = lax.axis_index("s")
    w = c * _NS + s
    row_base = s * _ROWS_PER_TILE
    ci = pltpu.async_copy(dst_hbm.at[w, pl.ds(0, _BLK_CH)], didx, sem)
    pltpu.sync_copy(zeros_hbm.at[pl.ds(row_base, _ROWS_PER_TILE)],
                    acc.at[pl.ds(row_base, _ROWS_PER_TILE)])
    pltpu.sync_copy(ones_hbm, ones_v)
    ci.wait()
    plsc.subcore_barrier()

    def block(j, carry):
        def step(i, c2):
            pltpu.sync_copy(ones_v, acc.at[didx.at[i]], add=True)
            return c2
        lax.fori_loop(0, _BLK_CH, step, 0)

        @pl.when(j + 1 < _NFULL)
        def _():
            j0 = pl.multiple_of((j + 1) * _BLK_CH, _BLK_CH)
            pltpu.sync_copy(dst_hbm.at[w, pl.ds(j0, _BLK_CH)], didx)
        return carry

    lax.fori_loop(0, _NFULL, block, 0)
    pltpu.sync_copy(dst_hbm.at[w, pl.ds(_NFULL * _BLK_CH, _TAIL)],
                    didx.at[pl.ds(0, _TAIL)])
    for i in range(_TAIL):
        pltpu.sync_copy(ones_v, acc.at[didx.at[i]], add=True)
    plsc.subcore_barrier()
    pltpu.sync_copy(acc.at[pl.ds(row_base, _ROWS_PER_TILE)],
                    out_hbm.at[c, pl.ds(row_base, _ROWS_PER_TILE)])


@functools.lru_cache(maxsize=None)
def _get_sc_degree():
    mesh = plsc.VectorSubcoreMesh(core_axis_name="c", subcore_axis_name="s")
    return pl.kernel(
        _sc_degree_body,
        out_type=jax.ShapeDtypeStruct((_NC, _N_PAD, _DEG_W), jnp.float32),
        mesh=mesh,
        scratch_types=[
            pltpu.VMEM((_BLK_CH, _CHUNK), jnp.int32),
            pltpu.VMEM((_CHUNK, _DEG_W), jnp.float32),
            pltpu.VMEM_SHARED((_N_PAD, _DEG_W), jnp.float32),
            pltpu.SemaphoreType.DMA,
        ],
    )


def _sc_scatter_body(hp_hbm, src_hbm, dst_hbm, zeros_hbm, out_hbm,
                     sidx, didx, rows0, rows1, acc, sem0, sem1, isem):
    c = lax.axis_index("c")
    s = lax.axis_index("s")
    w = c * _NS + s
    row_base = s * _ROWS_PER_TILE
    ci = pltpu.async_copy(src_hbm.at[w, pl.ds(0, _BLK_CH)], sidx, isem)
    cj = pltpu.async_copy(dst_hbm.at[w, pl.ds(0, _BLK_CH)], didx, isem)
    pltpu.sync_copy(zeros_hbm.at[pl.ds(row_base, _ROWS_PER_TILE)],
                    acc.at[pl.ds(row_base, _ROWS_PER_TILE)])
    ci.wait()
    cj.wait()
    plsc.subcore_barrier()

    def process(nch):
        pltpu.async_copy(hp_hbm.at[sidx.at[0]], rows0, sem0)

        def pair(i, carry):
            k0 = 2 * i
            pltpu.async_copy(hp_hbm.at[sidx.at[k0 + 1]], rows1, sem1)
            pltpu.make_async_copy(hp_hbm.at[sidx.at[k0]], rows0, sem0).wait()
            pltpu.sync_copy(rows0, acc.at[didx.at[k0]], add=True)

            @pl.when(k0 + 2 < nch)
            def _():
                pltpu.async_copy(hp_hbm.at[sidx.at[k0 + 2]], rows0, sem0)

            pltpu.make_async_copy(hp_hbm.at[sidx.at[k0 + 1]], rows1, sem1).wait()
            pltpu.sync_copy(rows1, acc.at[didx.at[k0 + 1]], add=True)
            return carry

        lax.fori_loop(0, nch // 2, pair, 0)
        if nch % 2:
            pltpu.make_async_copy(hp_hbm.at[sidx.at[nch - 1]], rows0, sem0).wait()
            pltpu.sync_copy(rows0, acc.at[didx.at[nch - 1]], add=True)

    def blk(j, carry):
        process(_BLK_CH)

        @pl.when(j + 1 < _NFULL)
        def _():
            j0 = pl.multiple_of((j + 1) * _BLK_CH, _BLK_CH)
            pltpu.sync_copy(src_hbm.at[w, pl.ds(j0, _BLK_CH)], sidx)
            pltpu.sync_copy(dst_hbm.at[w, pl.ds(j0, _BLK_CH)], didx)
        return carry

    lax.fori_loop(0, _NFULL, blk, 0)
    pltpu.sync_copy(src_hbm.at[w, pl.ds(_NFULL * _BLK_CH, _TAIL)],
                    sidx.at[pl.ds(0, _TAIL)])
    pltpu.sync_copy(dst_hbm.at[w, pl.ds(_NFULL * _BLK_CH, _TAIL)],
                    didx.at[pl.ds(0, _TAIL)])
    process(_TAIL)

    plsc.subcore_barrier()
    pltpu.sync_copy(acc.at[pl.ds(row_base, _ROWS_PER_TILE)],
                    out_hbm.at[c, pl.ds(row_base, _ROWS_PER_TILE)])


@functools.lru_cache(maxsize=None)
def _get_sc_scatter():
    mesh = plsc.VectorSubcoreMesh(core_axis_name="c", subcore_axis_name="s")
    return pl.kernel(
        _sc_scatter_body,
        out_type=jax.ShapeDtypeStruct((_NC, _N_PAD, _D), jnp.float32),
        mesh=mesh,
        scratch_types=[
            pltpu.VMEM((_BLK_CH, _CHUNK), jnp.int32),
            pltpu.VMEM((_BLK_CH, _CHUNK), jnp.int32),
            pltpu.VMEM((_CHUNK, _D), jnp.float32),
            pltpu.VMEM((_CHUNK, _D), jnp.float32),
            pltpu.VMEM_SHARED((_N_PAD, _D), jnp.float32),
            pltpu.SemaphoreType.DMA,
            pltpu.SemaphoreType.DMA,
            pltpu.SemaphoreType.DMA,
        ],
    )


_BLK = 400
_GRID = _N // _BLK


def _tc_mm1_body(x_ref, w1_ref, h_ref):
    h_ref[...] = jnp.dot(x_ref[...], w1_ref[...], preferred_element_type=jnp.float32)


def _tc_scale1_body(degp_ref, h_ref, hp_ref, dinv_ref):
    deg = degp_ref[0, :, 0:1] + degp_ref[1, :, 0:1] + 1.0
    dinv = lax.rsqrt(deg)
    hp_ref[...] = dinv * h_ref[...]
    dinv_ref[...] = dinv


def _tc2_body(parts_ref, hp_ref, dinv_ref, b1_ref, w2_ref, hp2_ref):
    dinv = dinv_ref[...]
    agg = parts_ref[0] + parts_ref[1] + hp_ref[...]
    o1 = jnp.maximum(dinv * agg + b1_ref[...], 0.0)
    hp2_ref[...] = dinv * jnp.dot(o1, w2_ref[...], preferred_element_type=jnp.float32)


def _tc3_body(parts_ref, hp_ref, dinv_ref, b2_ref, wf1_ref, bf1_ref,
              wf2_ref, bf2_ref, y_ref):
    dinv = dinv_ref[...]
    agg = parts_ref[0] + parts_ref[1] + hp_ref[...]
    o2 = jnp.maximum(dinv * agg + b2_ref[...], 0.0)
    h3 = jnp.maximum(
        jnp.dot(o2, wf1_ref[...], preferred_element_type=jnp.float32) + bf1_ref[...],
        0.0)
    y_ref[...] = jnp.dot(h3, wf2_ref[...], preferred_element_type=jnp.float32) + bf2_ref[...]


def _row_blk(*trail):
    return pl.BlockSpec((_BLK,) + trail, lambda i: (i,) + (0,) * len(trail))


def _parts_blk(width):
    return pl.BlockSpec((_NC, _BLK, width), lambda i: (0, i, 0))


def _full(shape):
    return pl.BlockSpec(shape, lambda i: (0,) * len(shape))


_tc_mm1 = pl.pallas_call(
    _tc_mm1_body,
    grid=(_GRID,),
    in_specs=[_row_blk(_D), _full((_D, _D))],
    out_specs=_row_blk(_D),
    out_shape=jax.ShapeDtypeStruct((_N, _D), jnp.float32),
)

_tc_scale1 = pl.pallas_call(
    _tc_scale1_body,
    grid=(_GRID,),
    in_specs=[_parts_blk(_DEG_W), _row_blk(_D)],
    out_specs=[_row_blk(_D), _row_blk(1)],
    out_shape=[jax.ShapeDtypeStruct((_N, _D), jnp.float32),
               jax.ShapeDtypeStruct((_N, 1), jnp.float32)],
)

_tc2 = pl.pallas_call(
    _tc2_body,
    grid=(_GRID,),
    in_specs=[_parts_blk(_D), _row_blk(_D), _row_blk(1), _full((1, _D)),
              _full((_D, _D))],
    out_specs=_row_blk(_D),
    out_shape=jax.ShapeDtypeStruct((_N, _D), jnp.float32),
)

_tc3 = pl.pallas_call(
    _tc3_body,
    grid=(_GRID,),
    in_specs=[_parts_blk(_D), _row_blk(_D), _row_blk(1), _full((1, _D)),
              _full((_D, 64)), _full((1, 64)), _full((64, 1)), _full((1, 1))],
    out_specs=_row_blk(1),
    out_shape=jax.ShapeDtypeStruct((_N, 1), jnp.float32),
)


def kernel(x, edge_index, W1, b1, W2, b2, Wf1, bf1, Wf2, bf2):
    src3 = edge_index[0].reshape(_NC * _NS, _STEPS, _CHUNK)
    dst3 = edge_index[1].reshape(_NC * _NS, _STEPS, _CHUNK)
    zeros_deg = jnp.zeros((_N_PAD, _DEG_W), jnp.float32)
    zeros_big = jnp.zeros((_N_PAD, _D), jnp.float32)
    ones_rows = jnp.ones((_CHUNK, _DEG_W), jnp.float32)

    deg_parts = _get_sc_degree()(dst3, ones_rows, zeros_deg)
    h1 = _tc_mm1(x, W1)
    hp1, dinv = _tc_scale1(deg_parts, h1)
    a1 = _get_sc_scatter()(hp1, src3, dst3, zeros_big)
    hp2 = _tc2(a1, hp1, dinv, b1.reshape(1, _D), W2)
    a2 = _get_sc_scatter()(hp2, src3, dst3, zeros_big)
    y = _tc3(a2, hp2, dinv, b2.reshape(1, _D), Wf1, bf1.reshape(1, 64),
             Wf2, bf2.reshape(1, 1))
    return y


# speedup vs baseline: 33.2385x; 1.2427x over previous
import functools

import jax
import jax.numpy as jnp
from jax import lax
from jax.experimental import pallas as pl
from jax.experimental.pallas import tpu as pltpu
from jax.experimental.pallas import tpu_sc as plsc

_N = 10000
_E = 320000
_D = 128

_NC = 2
_NS = 16
_CHUNK = 80
_EDGES_PER_TILE = _E // (_NC * _NS)
_STEPS = _EDGES_PER_TILE // _CHUNK
_NBLK = 5
_BLK_CH = _STEPS // _NBLK
_N_PAD = 10240
_ROWS_PER_TILE = _N_PAD // _NS
_DEG_W = 8


def _sc_degree_body(dst_hbm, ones_hbm, zeros_hbm, out_hbm, didx, ones_v, acc,
                    d0, d1, isem):
    c = lax.axis_index("c")
    t = lax.axis_index("s")
    w = c * _NS + t
    row_base = t * _ROWS_PER_TILE
    ci = pltpu.async_copy(dst_hbm.at[w, 0], didx.at[pl.ds(0, _BLK_CH)], isem)
    pltpu.sync_copy(zeros_hbm.at[pl.ds(row_base, _ROWS_PER_TILE)],
                    acc.at[pl.ds(row_base, _ROWS_PER_TILE)])
    pltpu.sync_copy(ones_hbm, ones_v)
    ci.wait()
    plsc.subcore_barrier()

    def block(j, carry):
        jb = lax.rem(j, 2) * _BLK_CH

        @pl.when(j + 1 < _NBLK)
        def _():
            jb_n = lax.rem(j + 1, 2) * _BLK_CH
            pltpu.async_copy(dst_hbm.at[w, j + 1],
                             didx.at[pl.ds(jb_n, _BLK_CH)], isem)

        pltpu.async_copy(ones_v, acc.at[didx.at[jb]], d0)

        def pair(g, c2):
            k0 = 2 * g
            pltpu.async_copy(ones_v, acc.at[didx.at[jb + k0 + 1]], d1)
            pltpu.make_async_copy(ones_v, acc.at[didx.at[jb]], d0).wait()

            @pl.when(k0 + 2 < _BLK_CH)
            def _():
                pltpu.async_copy(ones_v, acc.at[didx.at[jb + k0 + 2]], d0)

            pltpu.make_async_copy(ones_v, acc.at[didx.at[jb]], d1).wait()
            return c2

        lax.fori_loop(0, _BLK_CH // 2, pair, 0)
        pltpu.make_async_copy(ones_v, acc.at[didx.at[jb]], d0).wait()

        @pl.when(j + 1 < _NBLK)
        def _():
            jb_n = lax.rem(j + 1, 2) * _BLK_CH
            pltpu.make_async_copy(dst_hbm.at[w, j + 1],
                                  didx.at[pl.ds(jb_n, _BLK_CH)], isem).wait()
        return carry

    lax.fori_loop(0, _NBLK, block, 0)
    plsc.subcore_barrier()
    pltpu.sync_copy(acc.at[pl.ds(row_base, _ROWS_PER_TILE)],
                    out_hbm.at[c, pl.ds(row_base, _ROWS_PER_TILE)])


@functools.lru_cache(maxsize=None)
def _get_sc_degree():
    mesh = plsc.VectorSubcoreMesh(core_axis_name="c", subcore_axis_name="s")
    return pl.kernel(
        _sc_degree_body,
        out_type=jax.ShapeDtypeStruct((_NC, _N_PAD, _DEG_W), jnp.float32),
        mesh=mesh,
        scratch_types=[
            pltpu.VMEM((2 * _BLK_CH, _CHUNK), jnp.int32),
            pltpu.VMEM((_CHUNK, _DEG_W), jnp.float32),
            pltpu.VMEM_SHARED((_N_PAD, _DEG_W), jnp.float32),
            pltpu.SemaphoreType.DMA,
            pltpu.SemaphoreType.DMA,
            pltpu.SemaphoreType.DMA,
        ],
    )


def _sc_scatter_body(hp_hbm, src_hbm, dst_hbm, zeros_hbm, out_hbm,
                     sidx, didx, rows0, rows1, rows2, acc,
                     g0, g1, g2, s0, s1, s2, isem):
    c = lax.axis_index("c")
    t = lax.axis_index("s")
    w = c * _NS + t
    row_base = t * _ROWS_PER_TILE
    ci = pltpu.async_copy(src_hbm.at[w, 0], sidx.at[pl.ds(0, _BLK_CH)], isem)
    cj = pltpu.async_copy(dst_hbm.at[w, 0], didx.at[pl.ds(0, _BLK_CH)], isem)
    pltpu.sync_copy(zeros_hbm.at[pl.ds(row_base, _ROWS_PER_TILE)],
                    acc.at[pl.ds(row_base, _ROWS_PER_TILE)])
    ci.wait()
    cj.wait()
    plsc.subcore_barrier()

    rows = (rows0, rows1, rows2)
    gsem = (g0, g1, g2)
    ssem = (s0, s1, s2)

    def idx_row(k):
        return lax.rem(k // _BLK_CH, 2) * _BLK_CH + lax.rem(k, _BLK_CH)

    def wait_gather(ph):
        pltpu.make_async_copy(hp_hbm.at[sidx.at[0]], rows[ph], gsem[ph]).wait()

    def wait_scatter(ph):
        pltpu.make_async_copy(rows[ph], acc.at[didx.at[0]], ssem[ph]).wait()

    def issue_gather(k, ph):
        pltpu.async_copy(hp_hbm.at[sidx.at[idx_row(k)]], rows[ph], gsem[ph])

    def step(k, ph):
        j = k // _BLK_CH
        p = lax.rem(k, _BLK_CH)
        bn = (ph + 2) % 3

        @pl.when(k >= 1)
        def _():
            wait_scatter(bn)

        @pl.when((p == 0) & (j + 1 < _NBLK))
        def _():
            jb_n = lax.rem(j + 1, 2) * _BLK_CH
            pltpu.async_copy(src_hbm.at[w, j + 1],
                             sidx.at[pl.ds(jb_n, _BLK_CH)], isem)
            pltpu.async_copy(dst_hbm.at[w, j + 1],
                             didx.at[pl.ds(jb_n, _BLK_CH)], isem)

        @pl.when((p == _BLK_CH - 3) & (j + 1 < _NBLK))
        def _():
            pltpu.make_async_copy(src_hbm.at[w, 0],
                                  sidx.at[pl.ds(0, _BLK_CH)], isem).wait()
            pltpu.make_async_copy(dst_hbm.at[w, 0],
                                  didx.at[pl.ds(0, _BLK_CH)], isem).wait()

        @pl.when(k + 2 < _STEPS)
        def _():
            issue_gather(k + 2, bn)

        wait_gather(ph)
        pltpu.async_copy(rows[ph], acc.at[didx.at[idx_row(k)]], ssem[ph])

    issue_gather(jnp.int32(0), 0)
    issue_gather(jnp.int32(1), 1)

    def group(g, carry):
        k = 3 * g
        step(k, 0)
        step(k + 1, 1)
        step(k + 2, 2)
        return carry

    lax.fori_loop(0, _STEPS // 3, group, 0)
    step(jnp.int32(_STEPS - 2), 0)
    step(jnp.int32(_STEPS - 1), 1)
    wait_scatter(1)

    plsc.subcore_barrier()
    pltpu.sync_copy(acc.at[pl.ds(row_base, _ROWS_PER_TILE)],
                    out_hbm.at[c, pl.ds(row_base, _ROWS_PER_TILE)])


@functools.lru_cache(maxsize=None)
def _get_sc_scatter():
    mesh = plsc.VectorSubcoreMesh(core_axis_name="c", subcore_axis_name="s")
    return pl.kernel(
        _sc_scatter_body,
        out_type=jax.ShapeDtypeStruct((_NC, _N_PAD, _D), jnp.float32),
        mesh=mesh,
        scratch_types=[
            pltpu.VMEM((2 * _BLK_CH, _CHUNK), jnp.int32),
            pltpu.VMEM((2 * _BLK_CH, _CHUNK), jnp.int32),
            pltpu.VMEM((_CHUNK, _D), jnp.float32),
            pltpu.VMEM((_CHUNK, _D), jnp.float32),
            pltpu.VMEM((_CHUNK, _D), jnp.float32),
            pltpu.VMEM_SHARED((_N_PAD, _D), jnp.float32),
            pltpu.SemaphoreType.DMA,
            pltpu.SemaphoreType.DMA,
            pltpu.SemaphoreType.DMA,
            pltpu.SemaphoreType.DMA,
            pltpu.SemaphoreType.DMA,
            pltpu.SemaphoreType.DMA,
            pltpu.SemaphoreType.DMA,
        ],
    )


_BLK = 400
_GRID = _N // _BLK


def _tc_mm1_body(x_ref, w1_ref, h_ref):
    h_ref[...] = jnp.dot(x_ref[...], w1_ref[...], preferred_element_type=jnp.float32)


def _tc_scale1_body(degp_ref, h_ref, hp_ref, dinv_ref):
    deg = degp_ref[0, :, 0:1] + degp_ref[1, :, 0:1] + 1.0
    dinv = lax.rsqrt(deg)
    hp_ref[...] = dinv * h_ref[...]
    dinv_ref[...] = dinv


def _tc2_body(parts_ref, hp_ref, dinv_ref, b1_ref, w2_ref, hp2_ref):
    dinv = dinv_ref[...]
    agg = parts_ref[0] + parts_ref[1] + hp_ref[...]
    o1 = jnp.maximum(dinv * agg + b1_ref[...], 0.0)
    hp2_ref[...] = dinv * jnp.dot(o1, w2_ref[...], preferred_element_type=jnp.float32)


def _tc3_body(parts_ref, hp_ref, dinv_ref, b2_ref, wf1_ref, bf1_ref,
              wf2_ref, bf2_ref, y_ref):
    dinv = dinv_ref[...]
    agg = parts_ref[0] + parts_ref[1] + hp_ref[...]
    o2 = jnp.maximum(dinv * agg + b2_ref[...], 0.0)
    h3 = jnp.maximum(
        jnp.dot(o2, wf1_ref[...], preferred_element_type=jnp.float32) + bf1_ref[...],
        0.0)
    y_ref[...] = jnp.dot(h3, wf2_ref[...], preferred_element_type=jnp.float32) + bf2_ref[...]


def _row_blk(*trail):
    return pl.BlockSpec((_BLK,) + trail, lambda i: (i,) + (0,) * len(trail))


def _parts_blk(width):
    return pl.BlockSpec((_NC, _BLK, width), lambda i: (0, i, 0))


def _full(shape):
    return pl.BlockSpec(shape, lambda i: (0,) * len(shape))


_tc_mm1 = pl.pallas_call(
    _tc_mm1_body,
    grid=(_GRID,),
    in_specs=[_row_blk(_D), _full((_D, _D))],
    out_specs=_row_blk(_D),
    out_shape=jax.ShapeDtypeStruct((_N, _D), jnp.float32),
)

_tc_scale1 = pl.pallas_call(
    _tc_scale1_body,
    grid=(_GRID,),
    in_specs=[_parts_blk(_DEG_W), _row_blk(_D)],
    out_specs=[_row_blk(_D), _row_blk(1)],
    out_shape=[jax.ShapeDtypeStruct((_N, _D), jnp.float32),
               jax.ShapeDtypeStruct((_N, 1), jnp.float32)],
)

_tc2 = pl.pallas_call(
    _tc2_body,
    grid=(_GRID,),
    in_specs=[_parts_blk(_D), _row_blk(_D), _row_blk(1), _full((1, _D)),
              _full((_D, _D))],
    out_specs=_row_blk(_D),
    out_shape=jax.ShapeDtypeStruct((_N, _D), jnp.float32),
)

_tc3 = pl.pallas_call(
    _tc3_body,
    grid=(_GRID,),
    in_specs=[_parts_blk(_D), _row_blk(_D), _row_blk(1), _full((1, _D)),
              _full((_D, 64)), _full((1, 64)), _full((64, 1)), _full((1, 1))],
    out_specs=_row_blk(1),
    out_shape=jax.ShapeDtypeStruct((_N, 1), jnp.float32),
)


def kernel(x, edge_index, W1, b1, W2, b2, Wf1, bf1, Wf2, bf2):
    src4 = edge_index[0].reshape(_NC * _NS, _NBLK, _BLK_CH, _CHUNK)
    dst4 = edge_index[1].reshape(_NC * _NS, _NBLK, _BLK_CH, _CHUNK)
    zeros_deg = jnp.zeros((_N_PAD, _DEG_W), jnp.float32)
    zeros_big = jnp.zeros((_N_PAD, _D), jnp.float32)
    ones_rows = jnp.ones((_CHUNK, _DEG_W), jnp.float32)

    deg_parts = _get_sc_degree()(dst4, ones_rows, zeros_deg)
    h1 = _tc_mm1(x, W1)
    hp1, dinv = _tc_scale1(deg_parts, h1)
    a1 = _get_sc_scatter()(hp1, src4, dst4, zeros_big)
    hp2 = _tc2(a1, hp1, dinv, b1.reshape(1, _D), W2)
    a2 = _get_sc_scatter()(hp2, src4, dst4, zeros_big)
    y = _tc3(a2, hp2, dinv, b2.reshape(1, _D), Wf1, bf1.reshape(1, 64),
             Wf2, bf2.reshape(1, 1))
    return y
